# baseline (speedup 1.0000x reference)
import jax
import jax.numpy as jnp
from jax import lax
from jax.experimental import pallas as pl
from jax.experimental.pallas import tpu as pltpu
from jax.experimental.pallas import tpu_sc as plsc

BATCH = 16384
EMB = 32
_NC = 2
_NS = 16
_NW = _NC * _NS
_BPW = BATCH // _NW
_CHUNK = 128
_NCHUNK = _BPW // _CHUNK


def _emb_body(uids_hbm, iids_hbm, utab_hbm, itab_hbm,
              uout_hbm, iout_hbm,
              uidx, iidx, urows, irows, sem):
    wid = lax.axis_index("s") * _NC + lax.axis_index("c")
    pltpu.sync_copy(uids_hbm.at[wid], uidx)
    pltpu.sync_copy(iids_hbm.at[wid], iidx)
    waits = []
    for j in range(_NCHUNK):
        waits.append(pltpu.async_copy(
            utab_hbm.at[uidx.at[j]],
            urows.at[pl.ds(j * _CHUNK, _CHUNK)], sem))
        waits.append(pltpu.async_copy(
            itab_hbm.at[iidx.at[j]],
            irows.at[pl.ds(j * _CHUNK, _CHUNK)], sem))
    for w in waits:
        w.wait()
    pltpu.sync_copy(urows, uout_hbm.at[wid])
    pltpu.sync_copy(irows, iout_hbm.at[wid])


def _emb_call(uids, iids, user_table, item_table):
    mesh = plsc.VectorSubcoreMesh(core_axis_name="c", subcore_axis_name="s")
    f = pl.kernel(
        _emb_body, mesh=mesh,
        out_type=(
            jax.ShapeDtypeStruct((_NW, _BPW, EMB), jnp.float32),
            jax.ShapeDtypeStruct((_NW, _BPW, EMB), jnp.float32),
        ),
        scratch_types=[
            pltpu.VMEM((_NCHUNK, _CHUNK), jnp.int32),
            pltpu.VMEM((_NCHUNK, _CHUNK), jnp.int32),
            pltpu.VMEM((_BPW, EMB), jnp.float32),
            pltpu.VMEM((_BPW, EMB), jnp.float32),
            pltpu.SemaphoreType.DMA,
        ],
        compiler_params=pltpu.CompilerParams(use_tc_tiling_on_sc=False),
    )
    return f(uids, iids, user_table, item_table)


def kernel(user_ids, item_ids, user_table, item_table):
    uids = user_ids.astype(jnp.int32).reshape(_NW, _NCHUNK, _CHUNK)
    iids = item_ids.astype(jnp.int32).reshape(_NW, _NCHUNK, _CHUNK)
    uout, iout = _emb_call(uids, iids, user_table, item_table)
    return uout.reshape(BATCH, EMB), iout.reshape(BATCH, EMB)
